# Initial kernel scaffold; baseline (speedup 1.0000x reference)
import jax
import jax.numpy as jnp
from jax.experimental import pallas as pl


def kernel(ctrl_pts, Nu_uv, Nv_uv, uspan_uv, vspan_uv):
    raise NotImplementedError("write your pallas kernel here")


# speedup vs baseline: 5.8097x; 5.8097x over previous
import jax
import jax.numpy as jnp
from jax import lax
from jax.experimental import pallas as pl
from jax.experimental.pallas import tpu as pltpu

_P = 3
_Q = 3
_OUT_U = 256
_OUT_V = 256
_DIM = 3


def _surf_kernel(ctrl_ref, nu_ref, ub_ref, vb4_ref, nv4_ref,
                 out_ref, au_scr, b8_scr):
    b = pl.program_id(0)

    @pl.when(b == 0)
    def _build():
        col = lax.broadcasted_iota(jnp.int32, (_OUT_U, 64), 1)
        ub = ub_ref[...]
        au = jnp.zeros((_OUT_U, 64), jnp.float32)
        for l in range(_P + 1):
            au = au + jnp.where(col == ub + l, nu_ref[:, l:l + 1], 0.0)
        au_scr[...] = au

        row = lax.broadcasted_iota(jnp.int32, (256, 1024), 0)
        rn = row >> 2
        rc = row & 3
        ccol = lax.broadcasted_iota(jnp.int32, (256, 1024), 1) & 3
        vb4 = vb4_ref[...]
        num = jnp.zeros((256, 1024), jnp.float32)
        den = jnp.zeros((256, 1024), jnp.float32)
        for r in range(_Q + 1):
            nhit = rn == vb4 + r
            num = num + jnp.where((rc == ccol) & nhit, nv4_ref[r:r + 1, :], 0.0)
            den = den + jnp.where((rc == 3) & nhit, nv4_ref[r:r + 1, :], 0.0)
        b8_scr[:, :1024] = num
        b8_scr[:, 1024:] = den

    ctrl = ctrl_ref[0]
    t = lax.dot_general(au_scr[...], ctrl, (((1,), (0,)), ((), ())),
                        preferred_element_type=jnp.float32)
    r8 = lax.dot_general(t, b8_scr[...], (((1,), (0,)), ((), ())),
                         preferred_element_type=jnp.float32)
    out_ref[0] = r8[:, :1024] / r8[:, 1024:]


def kernel(ctrl_pts, Nu_uv, Nv_uv, uspan_uv, vspan_uv):
    batch, m, n, dimp1 = ctrl_pts.shape
    ctrl2 = ctrl_pts.reshape(batch, m, n * dimp1)

    ub_col = (uspan_uv - _P).astype(jnp.int32).reshape(_OUT_U, 1)
    vb4 = jnp.repeat((vspan_uv - _Q).astype(jnp.int32), dimp1).reshape(1, -1)
    nv4 = jnp.repeat(Nv_uv.astype(jnp.float32), dimp1, axis=0).T

    out4 = pl.pallas_call(
        _surf_kernel,
        grid=(batch,),
        in_specs=[
            pl.BlockSpec((1, m, n * dimp1), lambda b: (b, 0, 0)),
            pl.BlockSpec((_OUT_U, _P + 1), lambda b: (0, 0)),
            pl.BlockSpec((_OUT_U, 1), lambda b: (0, 0)),
            pl.BlockSpec((1, 1024), lambda b: (0, 0)),
            pl.BlockSpec((_Q + 1, 1024), lambda b: (0, 0)),
        ],
        out_specs=pl.BlockSpec((1, _OUT_U, 4 * _OUT_V), lambda b: (b, 0, 0)),
        out_shape=jax.ShapeDtypeStruct((batch, _OUT_U, 4 * _OUT_V),
                                       jnp.float32),
        scratch_shapes=[
            pltpu.VMEM((_OUT_U, 64), jnp.float32),
            pltpu.VMEM((256, 2048), jnp.float32),
        ],
    )(ctrl2, Nu_uv, ub_col, vb4, nv4)
    return out4.reshape(batch, _OUT_U, _OUT_V, dimp1)[..., :_DIM]


# speedup vs baseline: 7.2696x; 1.2513x over previous
import jax
import jax.numpy as jnp
from jax import lax
from jax.experimental import pallas as pl
from jax.experimental.pallas import tpu as pltpu

_P = 3
_Q = 3
_OUT_U = 256
_OUT_V = 256
_DIM = 3


def _surf_kernel(ctrl_ref, nu_ref, ub_ref, vb3_ref, nv3_ref,
                 out_ref, au_scr, b6_scr):
    b = pl.program_id(0)

    @pl.when(b == 0)
    def _build():
        col = lax.broadcasted_iota(jnp.int32, (_OUT_U, 64), 1)
        ub = ub_ref[...]
        au = jnp.zeros((_OUT_U, 64), jnp.float32)
        for l in range(_P + 1):
            au = au + jnp.where(col == ub + l, nu_ref[:, l:l + 1], 0.0)
        au_scr[...] = au

        row = lax.broadcasted_iota(jnp.int32, (256, 768), 0)
        rn = row >> 2
        rc = row & 3
        ccol = lax.broadcasted_iota(jnp.int32, (256, 768), 1) % 3
        vb3 = vb3_ref[...]
        num = jnp.zeros((256, 768), jnp.float32)
        den = jnp.zeros((256, 768), jnp.float32)
        for r in range(_Q + 1):
            nhit = rn == vb3 + r
            num = num + jnp.where((rc == ccol) & nhit, nv3_ref[r:r + 1, :], 0.0)
            den = den + jnp.where((rc == 3) & nhit, nv3_ref[r:r + 1, :], 0.0)
        b6_scr[:, :768] = num
        b6_scr[:, 768:] = den

    ctrl = ctrl_ref[0]
    t = lax.dot_general(au_scr[...], ctrl, (((1,), (0,)), ((), ())),
                        preferred_element_type=jnp.float32)
    r6 = lax.dot_general(t, b6_scr[...], (((1,), (0,)), ((), ())),
                         preferred_element_type=jnp.float32)
    out_ref[0] = r6[:, :768] / r6[:, 768:]


def kernel(ctrl_pts, Nu_uv, Nv_uv, uspan_uv, vspan_uv):
    batch, m, n, dimp1 = ctrl_pts.shape
    ctrl2 = ctrl_pts.reshape(batch, m, n * dimp1)

    ub_col = (uspan_uv - _P).astype(jnp.int32).reshape(_OUT_U, 1)
    vb3 = jnp.repeat((vspan_uv - _Q).astype(jnp.int32), _DIM).reshape(1, -1)
    nv3 = jnp.repeat(Nv_uv.astype(jnp.float32), _DIM, axis=0).T

    out3 = pl.pallas_call(
        _surf_kernel,
        grid=(batch,),
        in_specs=[
            pl.BlockSpec((1, m, n * dimp1), lambda b: (b, 0, 0)),
            pl.BlockSpec((_OUT_U, _P + 1), lambda b: (0, 0)),
            pl.BlockSpec((_OUT_U, 1), lambda b: (0, 0)),
            pl.BlockSpec((1, 768), lambda b: (0, 0)),
            pl.BlockSpec((_Q + 1, 768), lambda b: (0, 0)),
        ],
        out_specs=pl.BlockSpec((1, _OUT_U, _DIM * _OUT_V), lambda b: (b, 0, 0)),
        out_shape=jax.ShapeDtypeStruct((batch, _OUT_U, _DIM * _OUT_V),
                                       jnp.float32),
        scratch_shapes=[
            pltpu.VMEM((_OUT_U, 64), jnp.float32),
            pltpu.VMEM((256, 1536), jnp.float32),
        ],
    )(ctrl2, Nu_uv, ub_col, vb3, nv3)
    return out3.reshape(batch, _OUT_U, _OUT_V, _DIM)
